# baseline (speedup 1.0000x reference)
import functools

import jax
import jax.numpy as jnp
from jax import lax
from jax.experimental import pallas as pl
from jax.experimental.pallas import tpu as pltpu
from jax.experimental.pallas import tpu_sc as plsc

_L = 16
_UNROLL = 8


@functools.cache
def _make_sc_cumsum(R, C):
    info = plsc.get_sparse_core_info()
    NC, NS = info.num_cores, info.num_subcores
    NW = NC * NS
    rows_per_w = R // NW
    n_groups = rows_per_w // _L
    mesh = plsc.VectorSubcoreMesh(core_axis_name="c", subcore_axis_name="s")

    @functools.partial(
        pl.kernel,
        mesh=mesh,
        out_type=jax.ShapeDtypeStruct((R, C), jnp.float32),
        scratch_types=[pltpu.VMEM((_L, C), jnp.float32)],
        compiler_params=pltpu.CompilerParams(
            use_tc_tiling_on_sc=False, needs_layout_passes=False
        ),
    )
    def body(x_hbm, out_hbm, buf):
        wid = lax.axis_index("s") * NC + lax.axis_index("c")
        lane = lax.iota(jnp.int32, _L)

        def group(g, carry):
            base = wid * rows_per_w + g * _L
            pltpu.sync_copy(x_hbm.at[pl.ds(base, _L)], buf)

            def col(j, acc):
                jb = jnp.full((_L,), j * _UNROLL, dtype=jnp.int32)
                for k in range(_UNROLL):
                    jv = jb + k
                    v = plsc.load_gather(buf, [lane, jv])
                    acc = acc + v
                    plsc.store_scatter(buf, [lane, jv], acc)
                return acc

            lax.fori_loop(0, C // _UNROLL, col, jnp.zeros((_L,), jnp.float32))
            pltpu.sync_copy(buf, out_hbm.at[pl.ds(base, _L)])
            return carry

        lax.fori_loop(0, n_groups, group, 0)

    return body


def kernel(x):
    B, S, C = x.shape
    xf = x.reshape(B * S, C)
    out = _make_sc_cumsum(B * S, C)(xf)
    return out.reshape(x.shape)


# speedup vs baseline: 2.3715x; 2.3715x over previous
import functools

import jax
import jax.numpy as jnp
from jax import lax
from jax.experimental import pallas as pl
from jax.experimental.pallas import tpu as pltpu
from jax.experimental.pallas import tpu_sc as plsc

_L = 16
_UNROLL = 8


@functools.cache
def _make_sc_cumsum(R, C):
    info = plsc.get_sparse_core_info()
    NC, NS = info.num_cores, info.num_subcores
    NW = NC * NS
    rows_per_w = R // NW
    n_groups = rows_per_w // _L
    mesh = plsc.VectorSubcoreMesh(core_axis_name="c", subcore_axis_name="s")

    @functools.partial(
        pl.kernel,
        mesh=mesh,
        out_type=jax.ShapeDtypeStruct((R, C), jnp.float32),
        scratch_types=[pltpu.VMEM((_L, C), jnp.float32)],
        compiler_params=pltpu.CompilerParams(
            use_tc_tiling_on_sc=False, needs_layout_passes=False
        ),
    )
    def body(x_hbm, out_hbm, buf):
        wid = lax.axis_index("s") * NC + lax.axis_index("c")
        lane = lax.iota(jnp.int32, _L)

        def group(g, carry):
            base = wid * rows_per_w + g * _L
            pltpu.sync_copy(x_hbm.at[pl.ds(base, _L)], buf)

            acc = jnp.zeros((_L,), jnp.float32)
            for j in range(_L):
                jv = jnp.full((_L,), j, dtype=jnp.int32) - lane
                m = lane <= j
                v = plsc.load_gather(buf, [lane, jv], mask=m)
                acc = acc + jnp.where(m, v, 0.0)
                plsc.store_scatter(buf, [lane, jv], acc, mask=m)

            def col(i, carry):
                acc, jv = carry
                for _ in range(_UNROLL):
                    v = plsc.load_gather(buf, [lane, jv])
                    acc = acc + v
                    plsc.store_scatter(buf, [lane, jv], acc)
                    jv = jv + 1
                return acc, jv

            jv0 = jnp.full((_L,), _L, dtype=jnp.int32) - lane
            acc, _ = lax.fori_loop(
                0, (C - _L) // _UNROLL, col, (acc, jv0)
            )

            for j in range(C, C + _L - 1):
                jv = jnp.full((_L,), j, dtype=jnp.int32) - lane
                m = lane > (j - C)
                v = plsc.load_gather(buf, [lane, jv], mask=m)
                acc = acc + jnp.where(m, v, 0.0)
                plsc.store_scatter(buf, [lane, jv], acc, mask=m)

            pltpu.sync_copy(buf, out_hbm.at[pl.ds(base, _L)])
            return carry

        lax.fori_loop(0, n_groups, group, 0)

    return body


def kernel(x):
    B, S, C = x.shape
    xf = x.reshape(B * S, C)
    out = _make_sc_cumsum(B * S, C)(xf)
    return out.reshape(x.shape)


# speedup vs baseline: 2.7390x; 1.1550x over previous
import functools

import jax
import jax.numpy as jnp
from jax import lax
from jax.experimental import pallas as pl
from jax.experimental.pallas import tpu as pltpu
from jax.experimental.pallas import tpu_sc as plsc

_L = 16
_UNROLL = 8
_W = 1024
_D = 4


@functools.cache
def _make_sc_cumsum(R, C):
    info = plsc.get_sparse_core_info()
    NC, NS = info.num_cores, info.num_subcores
    NW = NC * NS
    rows_per_w = R // NW
    n_groups = rows_per_w // _L
    n_slabs = C // _W
    assert n_slabs == _D
    mesh = plsc.VectorSubcoreMesh(core_axis_name="c", subcore_axis_name="s")

    @functools.partial(
        pl.kernel,
        mesh=mesh,
        out_type=jax.ShapeDtypeStruct((R, C), jnp.float32),
        scratch_types=(
            [pltpu.VMEM((_L, _W), jnp.float32) for _ in range(_D)]
            + [pltpu.SemaphoreType.DMA for _ in range(2 * _D)]
        ),
        compiler_params=pltpu.CompilerParams(
            use_tc_tiling_on_sc=False, needs_layout_passes=False
        ),
    )
    def body(x_hbm, out_hbm, *bufs_and_sems):
        bufs = bufs_and_sems[:_D]
        in_sems = bufs_and_sems[_D : 2 * _D]
        out_sems = bufs_and_sems[2 * _D : 3 * _D]
        wid = lax.axis_index("s") * NC + lax.axis_index("c")
        lane = lax.iota(jnp.int32, _L)
        row0 = wid * rows_per_w

        def in_copy(g, h):
            b = h % _D
            return pltpu.make_async_copy(
                x_hbm.at[pl.ds(row0 + g * _L, _L), pl.ds(h * _W, _W)],
                bufs[b],
                in_sems[b],
            )

        def out_copy(g, h):
            b = h % _D
            return pltpu.make_async_copy(
                bufs[b],
                out_hbm.at[pl.ds(row0 + g * _L, _L), pl.ds(h * _W, _W)],
                out_sems[b],
            )

        def compute_slab(buf, acc):
            for j in range(_L):
                jv = jnp.full((_L,), j, dtype=jnp.int32) - lane
                m = lane <= j
                v = plsc.load_gather(buf, [lane, jv], mask=m)
                acc = acc + jnp.where(m, v, 0.0)
                plsc.store_scatter(buf, [lane, jv], acc, mask=m)

            def col(i, carry):
                acc, jv = carry
                for _ in range(_UNROLL):
                    v = plsc.load_gather(buf, [lane, jv])
                    acc = acc + v
                    plsc.store_scatter(buf, [lane, jv], acc)
                    jv = jv + 1
                return acc, jv

            jv0 = jnp.full((_L,), _L, dtype=jnp.int32) - lane
            acc, _ = lax.fori_loop(0, (_W - _L) // _UNROLL, col, (acc, jv0))

            for j in range(_W, _W + _L - 1):
                jv = jnp.full((_L,), j, dtype=jnp.int32) - lane
                m = lane > (j - _W)
                v = plsc.load_gather(buf, [lane, jv], mask=m)
                acc = acc + jnp.where(m, v, 0.0)
                plsc.store_scatter(buf, [lane, jv], acc, mask=m)
            return acc

        in_copy(0, 0).start()
        in_copy(0, 1).start()

        def group(g, carry):
            acc = jnp.zeros((_L,), jnp.float32)
            for h in range(_D):
                if h < 2:
                    @pl.when(g > 0)
                    def _():
                        out_copy(g - 1, h + 2).wait()

                    in_copy(g, h + 2).start()
                else:
                    out_copy(g, h - 2).wait()

                    @pl.when(g < n_groups - 1)
                    def _():
                        in_copy(g + 1, h - 2).start()

                in_copy(g, h).wait()
                acc = compute_slab(bufs[h], acc)
                out_copy(g, h).start()
            return carry

        lax.fori_loop(0, n_groups, group, 0)
        out_copy(n_groups - 1, 2).wait()
        out_copy(n_groups - 1, 3).wait()

    return body


def kernel(x):
    B, S, C = x.shape
    xf = x.reshape(B * S, C)
    out = _make_sc_cumsum(B * S, C)(xf)
    return out.reshape(x.shape)


# speedup vs baseline: 4.8740x; 1.7795x over previous
import functools

import jax
import jax.numpy as jnp
from jax import lax
from jax.experimental import pallas as pl
from jax.experimental.pallas import tpu as pltpu
from jax.experimental.pallas import tpu_sc as plsc

_L = 16
_UNROLL = 8
_W = 1024
_D = 4


@functools.cache
def _make_sc_cumsum(R, C):
    info = plsc.get_sparse_core_info()
    NC, NS = info.num_cores, info.num_subcores
    NW = NC * NS
    rows_per_w = R // NW
    n_groups = rows_per_w // _L
    n_slabs = C // _W
    assert n_slabs == _D
    mesh = plsc.VectorSubcoreMesh(core_axis_name="c", subcore_axis_name="s")

    @functools.partial(
        pl.kernel,
        mesh=mesh,
        out_type=jax.ShapeDtypeStruct((R, C), jnp.float32),
        scratch_types=(
            [pltpu.VMEM((_L, _W), jnp.float32) for _ in range(_D)]
            + [pltpu.SemaphoreType.DMA for _ in range(2 * _D)]
        ),
        compiler_params=pltpu.CompilerParams(
            use_tc_tiling_on_sc=False, needs_layout_passes=False
        ),
    )
    def body(x_hbm, out_hbm, *bufs_and_sems):
        bufs = bufs_and_sems[:_D]
        in_sems = bufs_and_sems[_D : 2 * _D]
        out_sems = bufs_and_sems[2 * _D : 3 * _D]
        wid = lax.axis_index("s") * NC + lax.axis_index("c")
        lane = lax.iota(jnp.int32, _L)
        row0 = wid * rows_per_w

        def in_copy(g, h):
            b = h % _D
            return pltpu.make_async_copy(
                x_hbm.at[pl.ds(row0 + g * _L, _L), pl.ds(h * _W, _W)],
                bufs[b],
                in_sems[b],
            )

        def out_copy(g, h):
            b = h % _D
            return pltpu.make_async_copy(
                bufs[b],
                out_hbm.at[pl.ds(row0 + g * _L, _L), pl.ds(h * _W, _W)],
                out_sems[b],
            )

        def compute_slab(buf, acc):
            for j in range(_L):
                jv = jnp.full((_L,), j, dtype=jnp.int32) - lane
                m = lane <= j
                v = plsc.load_gather(buf, [lane, jv], mask=m)
                acc = acc + jnp.where(m, v, 0.0)
                plsc.store_scatter(buf, [lane, jv], acc, mask=m)

            def col(i, carry):
                acc, jv = carry
                v = plsc.load_gather(buf, [lane, jv])
                acc = acc + v
                plsc.store_scatter(buf, [lane, jv], acc)
                return acc, jv + 1

            jv0 = jnp.full((_L,), _L, dtype=jnp.int32) - lane
            acc, _ = plsc.parallel_loop(
                0, _W - _L, unroll=_UNROLL, carry=(acc, jv0)
            )(col)

            for j in range(_W, _W + _L - 1):
                jv = jnp.full((_L,), j, dtype=jnp.int32) - lane
                m = lane > (j - _W)
                v = plsc.load_gather(buf, [lane, jv], mask=m)
                acc = acc + jnp.where(m, v, 0.0)
                plsc.store_scatter(buf, [lane, jv], acc, mask=m)
            return acc

        in_copy(0, 0).start()
        in_copy(0, 1).start()

        def group(g, carry):
            acc = jnp.zeros((_L,), jnp.float32)
            for h in range(_D):
                if h < 2:
                    @pl.when(g > 0)
                    def _():
                        out_copy(g - 1, h + 2).wait()

                    in_copy(g, h + 2).start()
                else:
                    out_copy(g, h - 2).wait()

                    @pl.when(g < n_groups - 1)
                    def _():
                        in_copy(g + 1, h - 2).start()

                in_copy(g, h).wait()
                acc = compute_slab(bufs[h], acc)
                out_copy(g, h).start()
            return carry

        lax.fori_loop(0, n_groups, group, 0)
        out_copy(n_groups - 1, 2).wait()
        out_copy(n_groups - 1, 3).wait()

    return body


def kernel(x):
    B, S, C = x.shape
    xf = x.reshape(B * S, C)
    out = _make_sc_cumsum(B * S, C)(xf)
    return out.reshape(x.shape)


# speedup vs baseline: 4.8793x; 1.0011x over previous
import functools

import jax
import jax.numpy as jnp
from jax import lax
from jax.experimental import pallas as pl
from jax.experimental.pallas import tpu as pltpu
from jax.experimental.pallas import tpu_sc as plsc

_L = 16
_UNROLL = 16
_W = 1024
_D = 4


@functools.cache
def _make_sc_cumsum(R, C):
    info = plsc.get_sparse_core_info()
    NC, NS = info.num_cores, info.num_subcores
    NW = NC * NS
    rows_per_w = R // NW
    n_groups = rows_per_w // _L
    n_slabs = C // _W
    assert n_slabs == _D
    mesh = plsc.VectorSubcoreMesh(core_axis_name="c", subcore_axis_name="s")

    @functools.partial(
        pl.kernel,
        mesh=mesh,
        out_type=jax.ShapeDtypeStruct((R, C), jnp.float32),
        scratch_types=(
            [pltpu.VMEM((_L, _W), jnp.float32) for _ in range(_D)]
            + [pltpu.SemaphoreType.DMA for _ in range(2 * _D)]
        ),
        compiler_params=pltpu.CompilerParams(
            use_tc_tiling_on_sc=False, needs_layout_passes=False
        ),
    )
    def body(x_hbm, out_hbm, *bufs_and_sems):
        bufs = bufs_and_sems[:_D]
        in_sems = bufs_and_sems[_D : 2 * _D]
        out_sems = bufs_and_sems[2 * _D : 3 * _D]
        wid = lax.axis_index("s") * NC + lax.axis_index("c")
        lane = lax.iota(jnp.int32, _L)
        row0 = wid * rows_per_w

        def in_copy(g, h):
            b = h % _D
            return pltpu.make_async_copy(
                x_hbm.at[pl.ds(row0 + g * _L, _L), pl.ds(h * _W, _W)],
                bufs[b],
                in_sems[b],
            )

        def out_copy(g, h):
            b = h % _D
            return pltpu.make_async_copy(
                bufs[b],
                out_hbm.at[pl.ds(row0 + g * _L, _L), pl.ds(h * _W, _W)],
                out_sems[b],
            )

        def compute_slab(buf, acc):
            for j in range(_L):
                jv = jnp.full((_L,), j, dtype=jnp.int32) - lane
                m = lane <= j
                v = plsc.load_gather(buf, [lane, jv], mask=m)
                acc = acc + jnp.where(m, v, 0.0)
                plsc.store_scatter(buf, [lane, jv], acc, mask=m)

            def col(i, carry):
                acc, jv = carry
                v = plsc.load_gather(buf, [lane, jv])
                acc = acc + v
                plsc.store_scatter(buf, [lane, jv], acc)
                return acc, jv + 1

            jv0 = jnp.full((_L,), _L, dtype=jnp.int32) - lane
            acc, _ = plsc.parallel_loop(
                0, _W - _L, unroll=_UNROLL, carry=(acc, jv0)
            )(col)

            for j in range(_W, _W + _L - 1):
                jv = jnp.full((_L,), j, dtype=jnp.int32) - lane
                m = lane > (j - _W)
                v = plsc.load_gather(buf, [lane, jv], mask=m)
                acc = acc + jnp.where(m, v, 0.0)
                plsc.store_scatter(buf, [lane, jv], acc, mask=m)
            return acc

        in_copy(0, 0).start()
        in_copy(0, 1).start()

        def group(g, carry):
            acc = jnp.zeros((_L,), jnp.float32)
            for h in range(_D):
                if h < 2:
                    @pl.when(g > 0)
                    def _():
                        out_copy(g - 1, h + 2).wait()

                    in_copy(g, h + 2).start()
                else:
                    out_copy(g, h - 2).wait()

                    @pl.when(g < n_groups - 1)
                    def _():
                        in_copy(g + 1, h - 2).start()

                in_copy(g, h).wait()
                acc = compute_slab(bufs[h], acc)
                out_copy(g, h).start()
            return carry

        lax.fori_loop(0, n_groups, group, 0)
        out_copy(n_groups - 1, 2).wait()
        out_copy(n_groups - 1, 3).wait()

    return body


def kernel(x):
    B, S, C = x.shape
    xf = x.reshape(B * S, C)
    out = _make_sc_cumsum(B * S, C)(xf)
    return out.reshape(x.shape)
